# Initial kernel scaffold; baseline (speedup 1.0000x reference)
import jax
import jax.numpy as jnp
from jax.experimental import pallas as pl


def kernel(x, edge_index, edge_weight, W1, b1, W2, b2, Wl, bl):
    raise NotImplementedError("write your pallas kernel here")


# speedup vs baseline: 66.9209x; 66.9209x over previous
import jax
import jax.numpy as jnp
from jax import lax
from jax.experimental import pallas as pl

SEQ = 15
N = 3
E = 9
HID = 32
INCH = 512
CLS = 2


def _gcn_kernel(xt_ref, ei_ref, eit_ref, ew_ref, w1_ref, b1_ref, w2_ref,
                b2_ref, wl_ref, bl_ref, out_ref):
    f32 = jnp.float32
    ei = ei_ref[...]
    eit = eit_ref[...]
    ew_in = ew_ref[...]

    row_l = ei[0:1, :]
    col_l = ei[1:2, :]
    row_s = eit[:, 0:1]
    col_s = eit[:, 1:2]

    mask_l = (row_l != col_l).astype(f32)
    ew = ew_in * mask_l

    lane_n = lax.broadcasted_iota(jnp.int32, (E, N), 1)
    S = (col_s == lane_n).astype(f32)
    R = (row_s == lane_n).astype(f32)
    self9 = (row_s == col_s).astype(f32)
    Rsel = R * self9

    loop_w = jnp.ones((SEQ, N), f32)
    for e in range(E):
        sel = Rsel[e:e + 1, :]
        loop_w = loop_w * (1.0 - sel) + ew_in[:, e:e + 1] * sel

    deg = jnp.dot(ew, S, preferred_element_type=f32) + loop_w
    dinv = jnp.where(deg > 0, lax.rsqrt(deg), jnp.zeros_like(deg))

    j9 = lax.broadcasted_iota(jnp.int32, (1, N * N), 1)
    c_of_j = j9 // N
    r_of_j = j9 % N
    M_edge = ((col_s == c_of_j) & (row_s == r_of_j)).astype(f32)
    W_edges = jnp.dot(ew, M_edge, preferred_element_type=f32)

    n_row = lax.broadcasted_iota(jnp.int32, (N, N * N), 0)
    Cmap = (c_of_j == n_row).astype(f32)
    Rmap = (r_of_j == n_row).astype(f32)
    Pdiag = Cmap * Rmap
    loop9 = jnp.dot(loop_w, Pdiag, preferred_element_type=f32)
    dinv_c = jnp.dot(dinv, Cmap, preferred_element_type=f32)
    dinv_r = jnp.dot(dinv, Rmap, preferred_element_type=f32)
    A = dinv_c * dinv_r * (W_edges + loop9)

    H = lax.dot_general(xt_ref[...], w1_ref[...], (((1,), (1,)), ((), ())),
                        preferred_element_type=f32)
    b1 = b1_ref[...]
    h1 = []
    for c in range(N):
        acc = jnp.broadcast_to(b1, (SEQ, HID))
        for r in range(N):
            acc = acc + A[:, N * c + r:N * c + r + 1] * H[SEQ * r:SEQ * (r + 1), :]
        h1.append(jnp.maximum(acc, 0.0))

    G = lax.dot_general(jnp.concatenate(h1, axis=0), w2_ref[...],
                        (((1,), (1,)), ((), ())),
                        preferred_element_type=f32)
    b2 = b2_ref[...]
    inv_seq = f32(1.0 / SEQ)
    y = jnp.broadcast_to(bl_ref[...], (1, CLS))
    wl = wl_ref[...]
    for c in range(N):
        acc = jnp.broadcast_to(b2, (SEQ, HID))
        for r in range(N):
            acc = acc + A[:, N * c + r:N * c + r + 1] * G[SEQ * r:SEQ * (r + 1), :]
        pooled = jnp.sum(acc, axis=0, keepdims=True) * inv_seq
        y = y + lax.dot_general(pooled, wl[:, HID * c:HID * (c + 1)],
                                (((1,), (1,)), ((), ())),
                                preferred_element_type=f32)
    out_ref[...] = y


def kernel(x, edge_index, edge_weight, W1, b1, W2, b2, Wl, bl):
    xt = jnp.transpose(x, (1, 0, 2)).reshape(N * SEQ, INCH)
    out = pl.pallas_call(
        _gcn_kernel,
        out_shape=jax.ShapeDtypeStruct((1, CLS), jnp.float32),
    )(xt, edge_index, edge_index.T, edge_weight, W1, b1.reshape(1, HID),
      W2, b2.reshape(1, HID), Wl, bl.reshape(1, CLS))
    return out.reshape(CLS)


# speedup vs baseline: 84.1399x; 1.2573x over previous
import jax
import jax.numpy as jnp
from jax import lax
from jax.experimental import pallas as pl

SEQ = 15
N = 3
E = 9
HID = 32
INCH = 512
CLS = 2


def _gcn_kernel(x_ref, ei_ref, ew_ref, w1_ref, b1_ref, w2_ref, b2_ref,
                wl_ref, bl_ref, out_ref):
    f32 = jnp.float32
    ei = ei_ref[...]
    row_l = ei[0:1, :]
    col_l = ei[1:2, :]
    ew_in = ew_ref[...]

    self_l = (row_l == col_l).astype(f32)
    ew = ew_in * (1.0 - self_l)

    n_s = lax.broadcasted_iota(jnp.int32, (N, E), 0)
    S_T = (col_l == n_s).astype(f32)
    R_T = (row_l == n_s).astype(f32)
    Rsel = R_T * self_l

    dot_nn = lambda a, b: lax.dot_general(a, b, (((1,), (1,)), ((), ())),
                                          preferred_element_type=f32)
    loop_w = jnp.ones((SEQ, N), f32)
    for e in range(E):
        m_e = Rsel[:, e]
        loop_w = (loop_w * (1.0 - m_e)[None, :]
                  + m_e[None, :] * ew_in[:, e:e + 1])

    deg = dot_nn(ew, S_T) + loop_w
    dinv = jnp.where(deg > 0, lax.rsqrt(deg), jnp.zeros_like(deg))

    j_s = lax.broadcasted_iota(jnp.int32, (N * N, E), 0)
    M_edge_T = ((col_l == j_s // N) & (row_l == j_s % N)).astype(f32)
    W_edges = dot_nn(ew, M_edge_T)

    jj = lax.broadcasted_iota(jnp.int32, (N * N, N), 0)
    nn = lax.broadcasted_iota(jnp.int32, (N * N, N), 1)
    CmapT = ((jj // N) == nn).astype(f32)
    RmapT = ((jj % N) == nn).astype(f32)
    loop9 = dot_nn(loop_w, CmapT * RmapT)
    A = dot_nn(dinv, CmapT) * dot_nn(dinv, RmapT) * (W_edges + loop9)

    H_t = dot_nn(x_ref[...], w1_ref[...])
    q_s = lax.broadcasted_iota(jnp.int32, (N * SEQ, N * SEQ), 0)
    k_l = lax.broadcasted_iota(jnp.int32, (N * SEQ, N * SEQ), 1)
    P = (k_l == N * (q_s % SEQ) + q_s // SEQ).astype(f32)
    H = jnp.dot(P, H_t, preferred_element_type=f32)

    b1 = b1_ref[...].reshape(1, HID)
    h1 = []
    for c in range(N):
        acc = jnp.broadcast_to(b1, (SEQ, HID))
        for r in range(N):
            acc = acc + A[:, N * c + r:N * c + r + 1] * H[SEQ * r:SEQ * (r + 1), :]
        h1.append(jnp.maximum(acc, 0.0))

    G = dot_nn(jnp.concatenate(h1, axis=0), w2_ref[...])
    b2 = b2_ref[...].reshape(1, HID)
    wl = wl_ref[...]
    inv_seq = f32(1.0 / SEQ)
    y = jnp.broadcast_to(bl_ref[...].reshape(1, CLS), (1, CLS))
    for c in range(N):
        acc = jnp.broadcast_to(b2, (SEQ, HID))
        for r in range(N):
            acc = acc + A[:, N * c + r:N * c + r + 1] * G[SEQ * r:SEQ * (r + 1), :]
        pooled = jnp.sum(acc, axis=0, keepdims=True) * inv_seq
        y = y + dot_nn(pooled, wl[:, HID * c:HID * (c + 1)])
    out_ref[...] = y


def kernel(x, edge_index, edge_weight, W1, b1, W2, b2, Wl, bl):
    out = pl.pallas_call(
        _gcn_kernel,
        out_shape=jax.ShapeDtypeStruct((1, CLS), jnp.float32),
    )(x.reshape(N * SEQ, INCH), edge_index, edge_weight, W1, b1, W2, b2,
      Wl, bl)
    return out.reshape(CLS)
